# baseline (speedup 1.0000x reference)
import functools

import jax
import jax.numpy as jnp
from jax import lax
from jax.experimental import pallas as pl
from jax.experimental.pallas import tpu as pltpu
from jax.experimental.pallas import tpu_sc as plsc

B = 16384
D4 = 16
V4 = 352899
NC, NS = 2, 16
NW = NC * NS
BPW = B // NW
CHUNK = 128
NCHUNK = BPW // CHUNK

BB = 2048


def _sc_gather(table, idx3):
    mesh = plsc.VectorSubcoreMesh(core_axis_name="c", subcore_axis_name="s")

    @functools.partial(
        pl.kernel,
        out_type=jax.ShapeDtypeStruct((D4, B), jnp.float32),
        mesh=mesh,
        scratch_types=[
            pltpu.VMEM((NCHUNK, CHUNK), jnp.int32),
            pltpu.VMEM((BPW, D4), jnp.float32),
            pltpu.VMEM((D4, BPW), jnp.float32),
            pltpu.SemaphoreType.DMA,
        ],
        compiler_params=pltpu.CompilerParams(
            needs_layout_passes=False, use_tc_tiling_on_sc=True),
    )
    def gather_kernel(table_hbm, idx_hbm, out_hbm, idx_v, rows_v, outT_v, sem):
        wid = lax.axis_index("s") * NC + lax.axis_index("c")
        pair = wid // 2
        half = wid % 2
        pltpu.sync_copy(idx_hbm.at[pair, pl.ds(half * NCHUNK, NCHUNK)], idx_v)
        for g in range(BPW // 16):
            vec = idx_v[g // 8, pl.ds((g % 8) * 16, 16)]
            for k in range(16):
                pltpu.async_copy(table_hbm.at[vec[k]],
                                 rows_v.at[g * 16 + k], sem)
        pltpu.make_async_copy(
            table_hbm.at[pl.ds(0, BPW)], rows_v, sem).wait()
        iota = lax.iota(jnp.int32, 16)
        for g in range(BPW // 16):
            ivec = iota + g * 16
            for d in range(D4):
                dvec = jnp.full((16,), d, jnp.int32)
                outT_v[d, pl.ds(g * 16, 16)] = plsc.load_gather(
                    rows_v, [ivec, dvec])
        pltpu.sync_copy(outT_v, out_hbm.at[:, pl.ds(wid * BPW, BPW)])

    return gather_kernel(table, idx3)


def _dense_body(p1_ref, p2_ref, p5_ref, p3_ref, p4_ref, v4_ref,
                W1_ref, b1_ref, E2_ref, E3_ref,
                W5A_ref, W5B_ref, W5C_ref, b5_ref,
                W6A_ref, W6B_ref, b6_ref, out_ref):
    f32 = jnp.float32
    dn0 = (((0,), (0,)), ((), ()))
    dn01 = (((0,), (1,)), ((), ()))
    x = jnp.concatenate([p1_ref[...], p2_ref[...], p5_ref[...]], axis=0)
    v1T = jnp.tanh(lax.dot_general(W1_ref[...], x, dn0,
                                   preferred_element_type=f32) + b1_ref[...])
    T2T = lax.dot_general(W5B_ref[...], E2_ref[...], dn01,
                          preferred_element_type=f32)
    T3T = lax.dot_general(W5C_ref[...], E3_ref[...], dn01,
                          preferred_element_type=f32)
    TTT = jnp.concatenate([T2T, T3T], axis=1)
    lane = lax.broadcasted_iota(jnp.int32, (8, BB), 0)
    p34 = jnp.where(lane < 4, p3_ref[...], p4_ref[...] + 4)
    ohT = (p34 == lane).astype(f32)
    selT = jnp.dot(TTT, ohT, preferred_element_type=f32)
    v5T = jnp.tanh(lax.dot_general(W5A_ref[...], v1T, dn0,
                                   preferred_element_type=f32)
                   + selT + b5_ref[...])
    x6 = (lax.dot_general(v4_ref[...], W6A_ref[...], dn0,
                          preferred_element_type=f32)
          + lax.dot_general(v5T, W6B_ref[...], dn0,
                            preferred_element_type=f32))
    out_ref[...] = jnp.tanh(x6 + b6_ref[...])


def _tc_dense(p1r, p2r, p5r, p3, p4, v4, W1, b1c, E2, E3,
              W5A, W5B, W5C, b5c, W6A, W6B, b6r):
    grid = (B // BB,)
    row = lambda i: (0, i)
    col = lambda i: (i, 0)
    rep = lambda i: (0, 0)
    return pl.pallas_call(
        _dense_body,
        grid=grid,
        in_specs=[
            pl.BlockSpec((1, BB), row),
            pl.BlockSpec((1, BB), row),
            pl.BlockSpec((1, BB), row),
            pl.BlockSpec((1, BB), row),
            pl.BlockSpec((1, BB), row),
            pl.BlockSpec((D4, BB), row),
            pl.BlockSpec((3, 8), rep),
            pl.BlockSpec((8, 1), rep),
            pl.BlockSpec((4, 8), rep),
            pl.BlockSpec((4, 8), rep),
            pl.BlockSpec((8, 64), rep),
            pl.BlockSpec((8, 64), rep),
            pl.BlockSpec((8, 64), rep),
            pl.BlockSpec((64, 1), rep),
            pl.BlockSpec((16, 128), rep),
            pl.BlockSpec((64, 128), rep),
            pl.BlockSpec((1, 128), rep),
        ],
        out_specs=pl.BlockSpec((BB, 128), col),
        out_shape=jax.ShapeDtypeStruct((B, 128), jnp.float32),
        compiler_params=pltpu.CompilerParams(
            dimension_semantics=("arbitrary",)),
    )(p1r, p2r, p5r, p3, p4, v4, W1, b1c, E2, E3,
      W5A, W5B, W5C, b5c, W6A, W6B, b6r)


def kernel(p1, p2, p5, p3, p4, p6, W1, b1, E2, E3, E4, W5, b5, W6, b6):
    idx3 = p6.reshape(NW // 2, 8, CHUNK)
    v4 = _sc_gather(E4, idx3)
    return _tc_dense(
        p1.reshape(1, B), p2.reshape(1, B), p5.reshape(1, B), p3, p4, v4,
        W1, b1.reshape(8, 1), E2, E3,
        W5[0:8], W5[8:16], W5[16:24], b5.reshape(64, 1),
        W6[0:16], W6[16:80], b6.reshape(1, 128),
    )


# speedup vs baseline: 1.0533x; 1.0533x over previous
import functools

import jax
import jax.numpy as jnp
from jax import lax
from jax.experimental import pallas as pl
from jax.experimental.pallas import tpu as pltpu
from jax.experimental.pallas import tpu_sc as plsc

B = 16384
D4 = 16
V4 = 352899
NC, NS = 2, 16
NW = NC * NS
BPW = B // NW
CHUNK = 128
NCHUNK = BPW // CHUNK

BB = 4096


def _sc_gather(table, idx3):
    mesh = plsc.VectorSubcoreMesh(core_axis_name="c", subcore_axis_name="s")

    @functools.partial(
        pl.kernel,
        out_type=jax.ShapeDtypeStruct((B, D4), jnp.float32),
        mesh=mesh,
        scratch_types=[
            pltpu.VMEM((NCHUNK, CHUNK), jnp.int32),
            pltpu.VMEM((BPW, D4), jnp.float32),
            pltpu.SemaphoreType.DMA,
        ],
        compiler_params=pltpu.CompilerParams(
            needs_layout_passes=False, use_tc_tiling_on_sc=True),
    )
    def gather_kernel(table_hbm, idx_hbm, out_hbm, idx_v, out_v, sem):
        wid = lax.axis_index("s") * NC + lax.axis_index("c")
        pair = wid // 2
        half = wid % 2
        pltpu.sync_copy(idx_hbm.at[pair, pl.ds(half * NCHUNK, NCHUNK)], idx_v)
        for g in range(BPW // 16):
            vec = idx_v[g // 8, pl.ds((g % 8) * 16, 16)]
            for k in range(16):
                pltpu.async_copy(table_hbm.at[vec[k]],
                                 out_v.at[g * 16 + k], sem)
        pltpu.make_async_copy(
            out_hbm.at[pl.ds(wid * BPW, BPW)], out_v, sem).wait()
        pltpu.sync_copy(out_v, out_hbm.at[pl.ds(wid * BPW, BPW)])

    return gather_kernel(table, idx3)


def _dense_body(p1_ref, p2_ref, p5_ref, p3_ref, p4_ref, v4_ref,
                W1_ref, b1_ref, E2_ref, E3_ref,
                W5A_ref, W5B_ref, W5C_ref, b5_ref,
                W6A_ref, W6B_ref, b6_ref, out_ref):
    f32 = jnp.float32
    dn0 = (((0,), (0,)), ((), ()))
    dn01 = (((0,), (1,)), ((), ()))
    x = jnp.concatenate([p1_ref[...], p2_ref[...], p5_ref[...]], axis=0)
    v1T = jnp.tanh(lax.dot_general(W1_ref[...], x, dn0,
                                   preferred_element_type=f32) + b1_ref[...])
    T2T = lax.dot_general(W5B_ref[...], E2_ref[...], dn01,
                          preferred_element_type=f32)
    T3T = lax.dot_general(W5C_ref[...], E3_ref[...], dn01,
                          preferred_element_type=f32)
    TTT = jnp.concatenate([T2T, T3T], axis=1)
    lane = lax.broadcasted_iota(jnp.int32, (8, BB), 0)
    p34 = jnp.where(lane < 4, p3_ref[...], p4_ref[...] + 4)
    ohT = (p34 == lane).astype(f32)
    selT = jnp.dot(TTT, ohT, preferred_element_type=f32)
    v5T = jnp.tanh(lax.dot_general(W5A_ref[...], v1T, dn0,
                                   preferred_element_type=f32)
                   + selT + b5_ref[...])
    x6 = (jnp.dot(v4_ref[...], W6A_ref[...], preferred_element_type=f32)
          + lax.dot_general(v5T, W6B_ref[...], dn0,
                            preferred_element_type=f32))
    out_ref[...] = jnp.tanh(x6 + b6_ref[...])


def _tc_dense(p1r, p2r, p5r, p3, p4, v4, W1, b1c, E2, E3,
              W5A, W5B, W5C, b5c, W6A, W6B, b6r):
    grid = (B // BB,)
    row = lambda i: (0, i)
    col = lambda i: (i, 0)
    rep = lambda i: (0, 0)
    return pl.pallas_call(
        _dense_body,
        grid=grid,
        in_specs=[
            pl.BlockSpec((1, BB), row),
            pl.BlockSpec((1, BB), row),
            pl.BlockSpec((1, BB), row),
            pl.BlockSpec((1, BB), row),
            pl.BlockSpec((1, BB), row),
            pl.BlockSpec((BB, D4), col),
            pl.BlockSpec((3, 8), rep),
            pl.BlockSpec((8, 1), rep),
            pl.BlockSpec((4, 8), rep),
            pl.BlockSpec((4, 8), rep),
            pl.BlockSpec((8, 64), rep),
            pl.BlockSpec((8, 64), rep),
            pl.BlockSpec((8, 64), rep),
            pl.BlockSpec((64, 1), rep),
            pl.BlockSpec((16, 128), rep),
            pl.BlockSpec((64, 128), rep),
            pl.BlockSpec((1, 128), rep),
        ],
        out_specs=pl.BlockSpec((BB, 128), col),
        out_shape=jax.ShapeDtypeStruct((B, 128), jnp.float32),
        compiler_params=pltpu.CompilerParams(
            dimension_semantics=("arbitrary",)),
    )(p1r, p2r, p5r, p3, p4, v4, W1, b1c, E2, E3,
      W5A, W5B, W5C, b5c, W6A, W6B, b6r)


def kernel(p1, p2, p5, p3, p4, p6, W1, b1, E2, E3, E4, W5, b5, W6, b6):
    idx3 = p6.reshape(NW // 2, 8, CHUNK)
    v4 = _sc_gather(E4, idx3)
    return _tc_dense(
        p1.reshape(1, B), p2.reshape(1, B), p5.reshape(1, B), p3, p4, v4,
        W1, b1.reshape(8, 1), E2, E3,
        W5[0:8], W5[8:16], W5[16:24], b5.reshape(64, 1),
        W6[0:16], W6[16:80], b6.reshape(1, 128),
    )


# speedup vs baseline: 1.0615x; 1.0077x over previous
import functools

import jax
import jax.numpy as jnp
from jax import lax
from jax.experimental import pallas as pl
from jax.experimental.pallas import tpu as pltpu
from jax.experimental.pallas import tpu_sc as plsc

B = 16384
D4 = 16
V4 = 352899
NC, NS = 2, 16
NW = NC * NS
BPW = B // NW
CHUNK = 128
NCHUNK = BPW // CHUNK

BB = 8192


def _sc_gather(table, idx3):
    mesh = plsc.VectorSubcoreMesh(core_axis_name="c", subcore_axis_name="s")

    @functools.partial(
        pl.kernel,
        out_type=jax.ShapeDtypeStruct((B, D4), jnp.float32),
        mesh=mesh,
        scratch_types=[
            pltpu.VMEM((NCHUNK, CHUNK), jnp.int32),
            pltpu.VMEM((BPW, D4), jnp.float32),
            pltpu.SemaphoreType.DMA,
        ],
        compiler_params=pltpu.CompilerParams(
            needs_layout_passes=False, use_tc_tiling_on_sc=True),
    )
    def gather_kernel(table_hbm, idx_hbm, out_hbm, idx_v, out_v, sem):
        wid = lax.axis_index("s") * NC + lax.axis_index("c")
        pair = wid // 2
        half = wid % 2
        pltpu.sync_copy(idx_hbm.at[pair, pl.ds(half * NCHUNK, NCHUNK)], idx_v)
        for g in range(BPW // 16):
            vec = idx_v[g // 8, pl.ds((g % 8) * 16, 16)]
            for k in range(16):
                pltpu.async_copy(table_hbm.at[vec[k]],
                                 out_v.at[g * 16 + k], sem)
        pltpu.make_async_copy(
            out_hbm.at[pl.ds(wid * BPW, BPW)], out_v, sem).wait()
        pltpu.sync_copy(out_v, out_hbm.at[pl.ds(wid * BPW, BPW)])

    return gather_kernel(table, idx3)


def _dense_body(p1_ref, p2_ref, p5_ref, p3_ref, p4_ref, v4_ref,
                W1_ref, b1_ref, E2_ref, E3_ref,
                W5A_ref, W5B_ref, W5C_ref, b5_ref,
                W6A_ref, W6B_ref, b6_ref, out_ref):
    f32 = jnp.float32
    dn0 = (((0,), (0,)), ((), ()))
    dn01 = (((0,), (1,)), ((), ()))
    x = jnp.concatenate([p1_ref[...], p2_ref[...], p5_ref[...]], axis=0)
    v1T = jnp.tanh(lax.dot_general(W1_ref[...], x, dn0,
                                   preferred_element_type=f32) + b1_ref[...])
    T2T = lax.dot_general(W5B_ref[...], E2_ref[...], dn01,
                          preferred_element_type=f32)
    T3T = lax.dot_general(W5C_ref[...], E3_ref[...], dn01,
                          preferred_element_type=f32)
    TTT = jnp.concatenate([T2T, T3T], axis=1)
    lane = lax.broadcasted_iota(jnp.int32, (8, BB), 0)
    p34 = jnp.where(lane < 4, p3_ref[...], p4_ref[...] + 4)
    ohT = (p34 == lane).astype(f32)
    selT = jnp.dot(TTT, ohT, preferred_element_type=f32)
    v5T = jnp.tanh(lax.dot_general(W5A_ref[...], v1T, dn0,
                                   preferred_element_type=f32)
                   + selT + b5_ref[...])
    x6 = (jnp.dot(v4_ref[...], W6A_ref[...], preferred_element_type=f32)
          + lax.dot_general(v5T, W6B_ref[...], dn0,
                            preferred_element_type=f32))
    out_ref[...] = jnp.tanh(x6 + b6_ref[...])


def _tc_dense(p1r, p2r, p5r, p3, p4, v4, W1, b1c, E2, E3,
              W5A, W5B, W5C, b5c, W6A, W6B, b6r):
    grid = (B // BB,)
    row = lambda i: (0, i)
    col = lambda i: (i, 0)
    rep = lambda i: (0, 0)
    return pl.pallas_call(
        _dense_body,
        grid=grid,
        in_specs=[
            pl.BlockSpec((1, BB), row),
            pl.BlockSpec((1, BB), row),
            pl.BlockSpec((1, BB), row),
            pl.BlockSpec((1, BB), row),
            pl.BlockSpec((1, BB), row),
            pl.BlockSpec((BB, D4), col),
            pl.BlockSpec((3, 8), rep),
            pl.BlockSpec((8, 1), rep),
            pl.BlockSpec((4, 8), rep),
            pl.BlockSpec((4, 8), rep),
            pl.BlockSpec((8, 64), rep),
            pl.BlockSpec((8, 64), rep),
            pl.BlockSpec((8, 64), rep),
            pl.BlockSpec((64, 1), rep),
            pl.BlockSpec((16, 128), rep),
            pl.BlockSpec((64, 128), rep),
            pl.BlockSpec((1, 128), rep),
        ],
        out_specs=pl.BlockSpec((BB, 128), col),
        out_shape=jax.ShapeDtypeStruct((B, 128), jnp.float32),
        compiler_params=pltpu.CompilerParams(
            dimension_semantics=("arbitrary",)),
    )(p1r, p2r, p5r, p3, p4, v4, W1, b1c, E2, E3,
      W5A, W5B, W5C, b5c, W6A, W6B, b6r)


def kernel(p1, p2, p5, p3, p4, p6, W1, b1, E2, E3, E4, W5, b5, W6, b6):
    idx3 = p6.reshape(NW // 2, 8, CHUNK)
    v4 = _sc_gather(E4, idx3)
    return _tc_dense(
        p1.reshape(1, B), p2.reshape(1, B), p5.reshape(1, B), p3, p4, v4,
        W1, b1.reshape(8, 1), E2, E3,
        W5[0:8], W5[8:16], W5[16:24], b5.reshape(64, 1),
        W6[0:16], W6[16:80], b6.reshape(1, 128),
    )
